# baseline (speedup 1.0000x reference)
import jax
import jax.numpy as jnp
from jax.experimental import pallas as pl
from jax.experimental.pallas import tpu as pltpu

N = 10000
E = 320000
D = 128
G = 64
LS = 64
H = 128
C = 10
L = 3


def _leaky_relu(x):
    return jnp.where(x >= 0, x, 0.01 * x)


def _head_body(pooled_ref, ls_ref, linh_W_ref, linh_b_ref,
               cls_W1_ref, cls_b1_ref, cls_W2_ref, cls_b2_ref, out_ref):
    pooled = pooled_ref[...]
    ls_h = ls_ref[...] @ linh_W_ref[...] + linh_b_ref[...]
    W1 = cls_W1_ref[...]
    o1 = pooled @ W1[:D, :] + ls_h @ W1[D:, :] + cls_b1_ref[...]
    o1 = _leaky_relu(o1)
    out_ref[...] = o1 @ cls_W2_ref[...] + cls_b2_ref[...]


def _head(pooled, ls, linh_W, linh_b, cls_W1, cls_b1, cls_W2, cls_b2):
    return pl.pallas_call(
        _head_body,
        out_shape=jax.ShapeDtypeStruct((G, C), jnp.float32),
    )(pooled, ls, linh_W, linh_b.reshape(1, H), cls_W1, cls_b1.reshape(1, H),
      cls_W2, cls_b2.reshape(1, C))


def kernel(node_deg, edge_index, batch, ls, conv_W, conv_b, eps, bn_gamma,
           bn_beta, linh_W, linh_b, cls_W1, cls_b1, cls_W2, cls_b2):
    eye = jnp.eye(D, dtype=jnp.float32)
    x = jnp.take(eye, node_deg, axis=0)
    src = edge_index[0]
    dst = edge_index[1]
    for l in range(L):
        msgs = jnp.take(x, src, axis=0)
        agg = jnp.zeros_like(x).at[dst].add(msgs)
        h = (1.0 + eps[l]) * x + agg
        h = h @ conv_W[l] + conv_b[l]
        mean = jnp.mean(h, axis=0)
        var = jnp.var(h, axis=0)
        h = (h - mean) / jnp.sqrt(var + 1e-5) * bn_gamma[l] + bn_beta[l]
        x = _leaky_relu(h)
    pooled = jax.ops.segment_sum(x, batch, num_segments=G)
    return _head(pooled, ls, linh_W, linh_b, cls_W1, cls_b1, cls_W2, cls_b2)


# speedup vs baseline: 4.8873x; 4.8873x over previous
import functools

import jax
import jax.numpy as jnp
from jax import lax
from jax.experimental import pallas as pl
from jax.experimental.pallas import tpu as pltpu
from jax.experimental.pallas import tpu_sc as plsc

N = 10000
E = 320000
D = 128
G = 64
LS = 64
H = 128
C = 10
L = 3

_NC = 2
_NS = 16
_NW = _NC * _NS
_KC = 80
_EPW = E // _NW
_NECH = _EPW // _KC
_NRCH = N // _KC
_RJ = -(-_NRCH // _NW)
_RJS = -(-_NRCH // _NS)

_RB = 1000
_NB = N // _RB


def _lrelu(x):
    return jnp.where(x >= 0, x, 0.01 * x)


def _sc_mesh():
    return plsc.VectorSubcoreMesh(core_axis_name="c", subcore_axis_name="s")


def _y0_body(deg_hbm, w_hbm, out_hbm, idx_v, rows_v, sem):
    c = lax.axis_index("c")
    s = lax.axis_index("s")
    wid = c * _NS + s
    for j in range(_RJ):
        ch = wid + j * _NW

        @pl.when(ch < _NRCH)
        def _():
            base = ch * _KC
            pltpu.sync_copy(deg_hbm.at[pl.ds(base, _KC)], idx_v)
            pltpu.async_copy(w_hbm.at[idx_v], rows_v, sem).wait()
            pltpu.sync_copy(rows_v, out_hbm.at[pl.ds(base, _KC)])


def _sc_y0(node_deg, w0):
    k = functools.partial(
        pl.kernel,
        out_type=jax.ShapeDtypeStruct((N, D), jnp.float32),
        mesh=_sc_mesh(),
        scratch_types=[
            pltpu.VMEM((_KC,), jnp.int32),
            pltpu.VMEM((_KC, D), jnp.float32),
            pltpu.SemaphoreType.DMA,
        ],
    )(_y0_body)
    return k(node_deg, w0)


def _msgpass_body(y_hbm, src_hbm, dst_hbm, zero_hbm, out_hbm,
                  src_v, dst_v, rows_v, acc_sh, sem):
    c = lax.axis_index("c")
    s = lax.axis_index("s")
    wid = c * _NS + s

    for j in range(_RJS):
        ch = s + j * _NS

        @pl.when(ch < _NRCH)
        def _():
            base = ch * _KC
            pltpu.sync_copy(zero_hbm.at[pl.ds(base, _KC)],
                            acc_sh.at[pl.ds(base, _KC)])
    plsc.subcore_barrier()

    wbase = wid * _EPW

    def body(i, carry):
        base = wbase + i * _KC
        pltpu.sync_copy(src_hbm.at[pl.ds(base, _KC)], src_v)
        pltpu.sync_copy(dst_hbm.at[pl.ds(base, _KC)], dst_v)
        pltpu.async_copy(y_hbm.at[src_v], rows_v, sem).wait()
        pltpu.sync_copy(rows_v, acc_sh.at[dst_v], add=True)
        return carry

    lax.fori_loop(0, _NECH, body, 0)
    plsc.subcore_barrier()

    for j in range(_RJS):
        ch = s + j * _NS

        @pl.when(ch < _NRCH)
        def _():
            base = ch * _KC
            pltpu.sync_copy(acc_sh.at[pl.ds(base, _KC)],
                            out_hbm.at[c, pl.ds(base, _KC)])


def _sc_msgpass(y, src, dst, zeros):
    k = functools.partial(
        pl.kernel,
        out_type=jax.ShapeDtypeStruct((_NC, N, D), jnp.float32),
        mesh=_sc_mesh(),
        scratch_types=[
            pltpu.VMEM((_KC,), jnp.int32),
            pltpu.VMEM((_KC,), jnp.int32),
            pltpu.VMEM((_KC, D), jnp.float32),
            pltpu.VMEM_SHARED((N, D), jnp.float32),
            pltpu.SemaphoreType.DMA,
        ],
    )(_msgpass_body)
    return k(y, src, dst, zeros)


def _pool_body(x_hbm, batch_hbm, zero_hbm, out_hbm, idx_v, rows_v, pool_sh, sem):
    c = lax.axis_index("c")
    s = lax.axis_index("s")
    wid = c * _NS + s

    @pl.when(s == 0)
    def _():
        pltpu.sync_copy(zero_hbm.at[pl.ds(0, G)], pool_sh)
    plsc.subcore_barrier()

    for j in range(_RJ):
        ch = wid + j * _NW

        @pl.when(ch < _NRCH)
        def _():
            base = ch * _KC
            pltpu.sync_copy(x_hbm.at[pl.ds(base, _KC)], rows_v)
            pltpu.sync_copy(batch_hbm.at[pl.ds(base, _KC)], idx_v)
            pltpu.sync_copy(rows_v, pool_sh.at[idx_v], add=True)
    plsc.subcore_barrier()

    @pl.when(s == 0)
    def _():
        pltpu.sync_copy(pool_sh, out_hbm.at[c])


def _sc_pool(x3, batch, zeros):
    k = functools.partial(
        pl.kernel,
        out_type=jax.ShapeDtypeStruct((_NC, G, D), jnp.float32),
        mesh=_sc_mesh(),
        scratch_types=[
            pltpu.VMEM((_KC,), jnp.int32),
            pltpu.VMEM((_KC, D), jnp.float32),
            pltpu.VMEM_SHARED((G, D), jnp.float32),
            pltpu.SemaphoreType.DMA,
        ],
    )(_pool_body)
    return k(x3, batch, zeros)


def _stats_body(y_ref, s_ref, e_ref, b_ref, h_ref, cs_ref, css_ref):
    i = pl.program_id(0)
    h = e_ref[...] * y_ref[...] + s_ref[0] + s_ref[1] + b_ref[...]
    h_ref[...] = h
    cs = jnp.sum(h, axis=0, keepdims=True)
    css = jnp.sum(h * h, axis=0, keepdims=True)

    @pl.when(i == 0)
    def _():
        cs_ref[...] = cs
        css_ref[...] = css

    @pl.when(i > 0)
    def _():
        cs_ref[...] += cs
        css_ref[...] += css


def _tc_stats(y, s2, epsp1, b):
    return pl.pallas_call(
        _stats_body,
        grid=(_NB,),
        in_specs=[
            pl.BlockSpec((_RB, D), lambda i: (i, 0)),
            pl.BlockSpec((_NC, _RB, D), lambda i: (0, i, 0)),
            pl.BlockSpec((1, D), lambda i: (0, 0)),
            pl.BlockSpec((1, D), lambda i: (0, 0)),
        ],
        out_specs=[
            pl.BlockSpec((_RB, D), lambda i: (i, 0)),
            pl.BlockSpec((1, D), lambda i: (0, 0)),
            pl.BlockSpec((1, D), lambda i: (0, 0)),
        ],
        out_shape=[
            jax.ShapeDtypeStruct((N, D), jnp.float32),
            jax.ShapeDtypeStruct((1, D), jnp.float32),
            jax.ShapeDtypeStruct((1, D), jnp.float32),
        ],
    )(y, s2, epsp1, b)


def _norm_body_mm(h_ref, cs_ref, css_ref, g_ref, bt_ref, w_ref, o_ref):
    mean = cs_ref[...] * (1.0 / N)
    var = css_ref[...] * (1.0 / N) - mean * mean
    inv = lax.rsqrt(var + 1e-5)
    xb = (h_ref[...] - mean) * (inv * g_ref[...]) + bt_ref[...]
    xb = _lrelu(xb)
    o_ref[...] = jnp.dot(xb, w_ref[...], preferred_element_type=jnp.float32)


def _norm_body(h_ref, cs_ref, css_ref, g_ref, bt_ref, o_ref):
    mean = cs_ref[...] * (1.0 / N)
    var = css_ref[...] * (1.0 / N) - mean * mean
    inv = lax.rsqrt(var + 1e-5)
    xb = (h_ref[...] - mean) * (inv * g_ref[...]) + bt_ref[...]
    o_ref[...] = _lrelu(xb)


def _tc_norm(h, cs, css, gamma, beta, w_next=None):
    vec_spec = pl.BlockSpec((1, D), lambda i: (0, 0))
    in_specs = [pl.BlockSpec((_RB, D), lambda i: (i, 0)),
                vec_spec, vec_spec, vec_spec, vec_spec]
    args = [h, cs, css, gamma, beta]
    body = _norm_body
    if w_next is not None:
        in_specs.append(pl.BlockSpec((D, D), lambda i: (0, 0)))
        args.append(w_next)
        body = _norm_body_mm
    return pl.pallas_call(
        body,
        grid=(_NB,),
        in_specs=in_specs,
        out_specs=pl.BlockSpec((_RB, D), lambda i: (i, 0)),
        out_shape=jax.ShapeDtypeStruct((N, D), jnp.float32),
    )(*args)


def _head_body(p_ref, ls_ref, lw_ref, lb_ref, w1_ref, b1_ref, w2_ref, b2_ref,
               out_ref):
    pooled = p_ref[0] + p_ref[1]
    ls_h = jnp.dot(ls_ref[...], lw_ref[...],
                   preferred_element_type=jnp.float32) + lb_ref[...]
    w1 = w1_ref[...]
    o1 = (jnp.dot(pooled, w1[:D, :], preferred_element_type=jnp.float32)
          + jnp.dot(ls_h, w1[D:, :], preferred_element_type=jnp.float32)
          + b1_ref[...])
    o1 = _lrelu(o1)
    out_ref[...] = jnp.dot(o1, w2_ref[...],
                           preferred_element_type=jnp.float32) + b2_ref[...]


def _tc_head(pooled2, ls, linh_W, linh_b, cls_W1, cls_b1, cls_W2, cls_b2):
    return pl.pallas_call(
        _head_body,
        out_shape=jax.ShapeDtypeStruct((G, C), jnp.float32),
    )(pooled2, ls, linh_W, linh_b.reshape(1, H), cls_W1,
      cls_b1.reshape(1, H), cls_W2, cls_b2.reshape(1, C))


def kernel(node_deg, edge_index, batch, ls, conv_W, conv_b, eps, bn_gamma,
           bn_beta, linh_W, linh_b, cls_W1, cls_b1, cls_W2, cls_b2):
    src = edge_index[0]
    dst = edge_index[1]
    zeros = jnp.zeros((N, D), jnp.float32)

    y = _sc_y0(node_deg.astype(jnp.int32), conv_W[0])
    for l in range(L):
        s2 = _sc_msgpass(y, src, dst, zeros)
        epsp1 = jnp.full((1, D), 1.0, jnp.float32) + eps[l]
        h, cs, css = _tc_stats(y, s2, epsp1, conv_b[l].reshape(1, D))
        w_next = conv_W[l + 1] if l + 1 < L else None
        y = _tc_norm(h, cs, css, bn_gamma[l].reshape(1, D),
                     bn_beta[l].reshape(1, D), w_next)

    pooled2 = _sc_pool(y, batch.astype(jnp.int32), zeros)
    return _tc_head(pooled2, ls, linh_W, linh_b, cls_W1, cls_b1, cls_W2,
                    cls_b2)


# speedup vs baseline: 11.4008x; 2.3327x over previous
import functools

import jax
import jax.numpy as jnp
from jax import lax
from jax.experimental import pallas as pl
from jax.experimental.pallas import tpu as pltpu
from jax.experimental.pallas import tpu_sc as plsc

N = 10000
E = 320000
D = 128
G = 64
LS = 64
H = 128
C = 10
L = 3

_NC = 2
_NS = 16
_NW = _NC * _NS
_EC = 125
_EPW = E // _NW
_NECH = _EPW // _EC
_PCH = 40
_KC = 80
_NRCH = N // _KC
_RJ = -(-_NRCH // _NW)
_RJS = -(-_NRCH // _NS)

_RB = 1000
_NB = N // _RB


def _lrelu(x):
    return jnp.where(x >= 0, x, 0.01 * x)


def _sc_mesh():
    return plsc.VectorSubcoreMesh(core_axis_name="c", subcore_axis_name="s")


def _y0_body(deg_hbm, w_hbm, out_hbm, idx_v, rows_v, sem):
    c = lax.axis_index("c")
    s = lax.axis_index("s")
    wid = c * _NS + s
    for j in range(_RJ):
        ch = wid + j * _NW

        @pl.when(ch < _NRCH)
        def _():
            base = ch * _KC
            pltpu.sync_copy(deg_hbm.at[pl.ds(base, _KC)], idx_v)
            pltpu.async_copy(w_hbm.at[idx_v], rows_v, sem).wait()
            pltpu.sync_copy(rows_v, out_hbm.at[pl.ds(base, _KC)])


def _sc_y0(node_deg, w0):
    k = functools.partial(
        pl.kernel,
        out_type=jax.ShapeDtypeStruct((N, D), jnp.float32),
        mesh=_sc_mesh(),
        scratch_types=[
            pltpu.VMEM((_KC,), jnp.int32),
            pltpu.VMEM((_KC, D), jnp.float32),
            pltpu.SemaphoreType.DMA,
        ],
    )(_y0_body)
    return k(node_deg, w0)


def _msgpass_body(y_hbm, srcr_hbm, dstr_hbm, zero_hbm, out_hbm,
                  src2_v, dst2_v, buf0, buf1, acc_sh, sem):
    c = lax.axis_index("c")
    s = lax.axis_index("s")
    wid = c * _NS + s

    for j in range(_RJS):
        ch = s + j * _NS

        @pl.when(ch < _NRCH)
        def _():
            base = ch * _KC
            pltpu.sync_copy(zero_hbm.at[pl.ds(base, _KC)],
                            acc_sh.at[pl.ds(base, _KC)])
    plsc.subcore_barrier()

    def gather(i, buf):
        pltpu.async_copy(y_hbm.at[src2_v.at[i]], buf, sem)

    def drain(i, buf):
        pltpu.make_async_copy(y_hbm.at[src2_v.at[i]], buf, sem).wait()

    def scat(i, buf):
        pltpu.sync_copy(buf, acc_sh.at[dst2_v.at[i]], add=True)

    for p in range(_NECH // _PCH):
        pltpu.sync_copy(
            srcr_hbm.at[pl.ds(wid * _NECH + p * _PCH, _PCH)], src2_v)
        pltpu.sync_copy(
            dstr_hbm.at[pl.ds(wid * _NECH + p * _PCH, _PCH)], dst2_v)

        gather(0, buf0)

        def body(i, carry):
            c0 = 2 * i
            gather(c0 + 1, buf1)
            drain(c0, buf0)
            scat(c0, buf0)

            @pl.when(c0 + 2 < _PCH)
            def _():
                gather(c0 + 2, buf0)
            drain(c0 + 1, buf1)
            scat(c0 + 1, buf1)
            return carry

        lax.fori_loop(0, _PCH // 2, body, 0)
    plsc.subcore_barrier()

    for j in range(_RJS):
        ch = s + j * _NS

        @pl.when(ch < _NRCH)
        def _():
            base = ch * _KC
            pltpu.sync_copy(acc_sh.at[pl.ds(base, _KC)],
                            out_hbm.at[c, pl.ds(base, _KC)])


def _sc_msgpass(y, src_r, dst_r, zeros):
    k = functools.partial(
        pl.kernel,
        out_type=jax.ShapeDtypeStruct((_NC, N, D), jnp.float32),
        mesh=_sc_mesh(),
        scratch_types=[
            pltpu.VMEM((_PCH, _EC), jnp.int32),
            pltpu.VMEM((_PCH, _EC), jnp.int32),
            pltpu.VMEM((_EC, D), jnp.float32),
            pltpu.VMEM((_EC, D), jnp.float32),
            pltpu.VMEM_SHARED((N, D), jnp.float32),
            pltpu.SemaphoreType.DMA,
        ],
    )(_msgpass_body)
    return k(y, src_r, dst_r, zeros)


def _pool_body(x_hbm, batch_hbm, zero_hbm, out_hbm, idx_v, rows_v, pool_sh, sem):
    c = lax.axis_index("c")
    s = lax.axis_index("s")
    wid = c * _NS + s

    @pl.when(s == 0)
    def _():
        pltpu.sync_copy(zero_hbm.at[pl.ds(0, G)], pool_sh)
    plsc.subcore_barrier()

    for j in range(_RJ):
        ch = wid + j * _NW

        @pl.when(ch < _NRCH)
        def _():
            base = ch * _KC
            pltpu.sync_copy(x_hbm.at[pl.ds(base, _KC)], rows_v)
            pltpu.sync_copy(batch_hbm.at[pl.ds(base, _KC)], idx_v)
            pltpu.sync_copy(rows_v, pool_sh.at[idx_v], add=True)
    plsc.subcore_barrier()

    @pl.when(s == 0)
    def _():
        pltpu.sync_copy(pool_sh, out_hbm.at[c])


def _sc_pool(x3, batch, zeros):
    k = functools.partial(
        pl.kernel,
        out_type=jax.ShapeDtypeStruct((_NC, G, D), jnp.float32),
        mesh=_sc_mesh(),
        scratch_types=[
            pltpu.VMEM((_KC,), jnp.int32),
            pltpu.VMEM((_KC, D), jnp.float32),
            pltpu.VMEM_SHARED((G, D), jnp.float32),
            pltpu.SemaphoreType.DMA,
        ],
    )(_pool_body)
    return k(x3, batch, zeros)


def _stats_body(y_ref, s_ref, e_ref, b_ref, h_ref, cs_ref, css_ref):
    i = pl.program_id(0)
    h = e_ref[...] * y_ref[...] + s_ref[0] + s_ref[1] + b_ref[...]
    h_ref[...] = h
    cs = jnp.sum(h, axis=0, keepdims=True)
    css = jnp.sum(h * h, axis=0, keepdims=True)

    @pl.when(i == 0)
    def _():
        cs_ref[...] = cs
        css_ref[...] = css

    @pl.when(i > 0)
    def _():
        cs_ref[...] += cs
        css_ref[...] += css


def _tc_stats(y, s2, epsp1, b):
    return pl.pallas_call(
        _stats_body,
        grid=(_NB,),
        in_specs=[
            pl.BlockSpec((_RB, D), lambda i: (i, 0)),
            pl.BlockSpec((_NC, _RB, D), lambda i: (0, i, 0)),
            pl.BlockSpec((1, D), lambda i: (0, 0)),
            pl.BlockSpec((1, D), lambda i: (0, 0)),
        ],
        out_specs=[
            pl.BlockSpec((_RB, D), lambda i: (i, 0)),
            pl.BlockSpec((1, D), lambda i: (0, 0)),
            pl.BlockSpec((1, D), lambda i: (0, 0)),
        ],
        out_shape=[
            jax.ShapeDtypeStruct((N, D), jnp.float32),
            jax.ShapeDtypeStruct((1, D), jnp.float32),
            jax.ShapeDtypeStruct((1, D), jnp.float32),
        ],
    )(y, s2, epsp1, b)


def _norm_body_mm(h_ref, cs_ref, css_ref, g_ref, bt_ref, w_ref, o_ref):
    mean = cs_ref[...] * (1.0 / N)
    var = css_ref[...] * (1.0 / N) - mean * mean
    inv = lax.rsqrt(var + 1e-5)
    xb = (h_ref[...] - mean) * (inv * g_ref[...]) + bt_ref[...]
    xb = _lrelu(xb)
    o_ref[...] = jnp.dot(xb, w_ref[...], preferred_element_type=jnp.float32)


def _norm_body(h_ref, cs_ref, css_ref, g_ref, bt_ref, o_ref):
    mean = cs_ref[...] * (1.0 / N)
    var = css_ref[...] * (1.0 / N) - mean * mean
    inv = lax.rsqrt(var + 1e-5)
    xb = (h_ref[...] - mean) * (inv * g_ref[...]) + bt_ref[...]
    o_ref[...] = _lrelu(xb)


def _tc_norm(h, cs, css, gamma, beta, w_next=None):
    vec_spec = pl.BlockSpec((1, D), lambda i: (0, 0))
    in_specs = [pl.BlockSpec((_RB, D), lambda i: (i, 0)),
                vec_spec, vec_spec, vec_spec, vec_spec]
    args = [h, cs, css, gamma, beta]
    body = _norm_body
    if w_next is not None:
        in_specs.append(pl.BlockSpec((D, D), lambda i: (0, 0)))
        args.append(w_next)
        body = _norm_body_mm
    return pl.pallas_call(
        body,
        grid=(_NB,),
        in_specs=in_specs,
        out_specs=pl.BlockSpec((_RB, D), lambda i: (i, 0)),
        out_shape=jax.ShapeDtypeStruct((N, D), jnp.float32),
    )(*args)


def _head_body(p_ref, ls_ref, lw_ref, lb_ref, w1_ref, b1_ref, w2_ref, b2_ref,
               out_ref):
    pooled = p_ref[0] + p_ref[1]
    ls_h = jnp.dot(ls_ref[...], lw_ref[...],
                   preferred_element_type=jnp.float32) + lb_ref[...]
    w1 = w1_ref[...]
    o1 = (jnp.dot(pooled, w1[:D, :], preferred_element_type=jnp.float32)
          + jnp.dot(ls_h, w1[D:, :], preferred_element_type=jnp.float32)
          + b1_ref[...])
    o1 = _lrelu(o1)
    out_ref[...] = jnp.dot(o1, w2_ref[...],
                           preferred_element_type=jnp.float32) + b2_ref[...]


def _tc_head(pooled2, ls, linh_W, linh_b, cls_W1, cls_b1, cls_W2, cls_b2):
    return pl.pallas_call(
        _head_body,
        out_shape=jax.ShapeDtypeStruct((G, C), jnp.float32),
    )(pooled2, ls, linh_W, linh_b.reshape(1, H), cls_W1,
      cls_b1.reshape(1, H), cls_W2, cls_b2.reshape(1, C))


def kernel(node_deg, edge_index, batch, ls, conv_W, conv_b, eps, bn_gamma,
           bn_beta, linh_W, linh_b, cls_W1, cls_b1, cls_W2, cls_b2):
    src_r = edge_index[0].reshape(E // _EC, _EC)
    dst_r = edge_index[1].reshape(E // _EC, _EC)
    zeros = jnp.zeros((N, D), jnp.float32)

    y = _sc_y0(node_deg.astype(jnp.int32), conv_W[0])
    for l in range(L):
        s2 = _sc_msgpass(y, src_r, dst_r, zeros)
        epsp1 = jnp.full((1, D), 1.0, jnp.float32) + eps[l]
        h, cs, css = _tc_stats(y, s2, epsp1, conv_b[l].reshape(1, D))
        w_next = conv_W[l + 1] if l + 1 < L else None
        y = _tc_norm(h, cs, css, bn_gamma[l].reshape(1, D),
                     bn_beta[l].reshape(1, D), w_next)

    pooled2 = _sc_pool(y, batch.astype(jnp.int32), zeros)
    return _tc_head(pooled2, ls, linh_W, linh_b, cls_W1, cls_b1, cls_W2,
                    cls_b2)


# speedup vs baseline: 11.4731x; 1.0063x over previous
import functools

import jax
import jax.numpy as jnp
from jax import lax
from jax.experimental import pallas as pl
from jax.experimental.pallas import tpu as pltpu
from jax.experimental.pallas import tpu_sc as plsc

N = 10000
E = 320000
D = 128
G = 64
LS = 64
H = 128
C = 10
L = 3

_NC = 2
_NS = 16
_NW = _NC * _NS
_EC = 125
_EPW = E // _NW
_NECH = _EPW // _EC
_PCH = 40
_KC = 80
_NRCH = N // _KC
_RJ = -(-_NRCH // _NW)
_RJS = -(-_NRCH // _NS)

_RB = 1000
_NB = N // _RB


def _lrelu(x):
    return jnp.where(x >= 0, x, 0.01 * x)


def _sc_mesh():
    return plsc.VectorSubcoreMesh(core_axis_name="c", subcore_axis_name="s")


def _y0_body(deg_hbm, w_hbm, out_hbm, idx_v, rows_v, sem):
    c = lax.axis_index("c")
    s = lax.axis_index("s")
    wid = c * _NS + s
    for j in range(_RJ):
        ch = wid + j * _NW

        @pl.when(ch < _NRCH)
        def _():
            base = ch * _KC
            pltpu.sync_copy(deg_hbm.at[pl.ds(base, _KC)], idx_v)
            pltpu.async_copy(w_hbm.at[idx_v], rows_v, sem).wait()
            pltpu.sync_copy(rows_v, out_hbm.at[pl.ds(base, _KC)])


def _sc_y0(node_deg, w0):
    k = functools.partial(
        pl.kernel,
        out_type=jax.ShapeDtypeStruct((N, D), jnp.float32),
        mesh=_sc_mesh(),
        scratch_types=[
            pltpu.VMEM((_KC,), jnp.int32),
            pltpu.VMEM((_KC, D), jnp.float32),
            pltpu.SemaphoreType.DMA,
        ],
    )(_y0_body)
    return k(node_deg, w0)


def _msgpass_body(y_hbm, srcr_hbm, dstr_hbm, zero_hbm, out_hbm,
                  src2_v, dst2_v, buf0, buf1, acc_sh, gsem, ssem):
    c = lax.axis_index("c")
    s = lax.axis_index("s")
    wid = c * _NS + s

    for j in range(_RJS):
        ch = s + j * _NS

        @pl.when(ch < _NRCH)
        def _():
            base = ch * _KC
            pltpu.sync_copy(zero_hbm.at[pl.ds(base, _KC)],
                            acc_sh.at[pl.ds(base, _KC)])
    plsc.subcore_barrier()

    def gather(i, buf):
        pltpu.async_copy(y_hbm.at[src2_v.at[i]], buf, gsem)

    def drain_g(i, buf):
        pltpu.make_async_copy(y_hbm.at[src2_v.at[i]], buf, gsem).wait()

    def scat(i, buf):
        pltpu.async_copy(buf, acc_sh.at[dst2_v.at[i]], ssem, add=True)

    def drain_s(i, buf):
        pltpu.make_async_copy(buf, acc_sh.at[dst2_v.at[i]], ssem).wait()

    for p in range(_NECH // _PCH):
        pltpu.sync_copy(
            srcr_hbm.at[pl.ds(wid * _NECH + p * _PCH, _PCH)], src2_v)
        pltpu.sync_copy(
            dstr_hbm.at[pl.ds(wid * _NECH + p * _PCH, _PCH)], dst2_v)

        gather(0, buf0)

        def body(i, carry):
            c0 = 2 * i

            @pl.when(i > 0)
            def _():
                drain_s(c0 - 1, buf1)
            gather(c0 + 1, buf1)
            drain_g(c0, buf0)
            scat(c0, buf0)

            drain_s(c0, buf0)

            @pl.when(c0 + 2 < _PCH)
            def _():
                gather(c0 + 2, buf0)
            drain_g(c0 + 1, buf1)
            scat(c0 + 1, buf1)
            return carry

        lax.fori_loop(0, _PCH // 2, body, 0)
        drain_s(_PCH - 1, buf1)
    plsc.subcore_barrier()

    for j in range(_RJS):
        ch = s + j * _NS

        @pl.when(ch < _NRCH)
        def _():
            base = ch * _KC
            pltpu.sync_copy(acc_sh.at[pl.ds(base, _KC)],
                            out_hbm.at[c, pl.ds(base, _KC)])


def _sc_msgpass(y, src_r, dst_r, zeros):
    k = functools.partial(
        pl.kernel,
        out_type=jax.ShapeDtypeStruct((_NC, N, D), jnp.float32),
        mesh=_sc_mesh(),
        scratch_types=[
            pltpu.VMEM((_PCH, _EC), jnp.int32),
            pltpu.VMEM((_PCH, _EC), jnp.int32),
            pltpu.VMEM((_EC, D), jnp.float32),
            pltpu.VMEM((_EC, D), jnp.float32),
            pltpu.VMEM_SHARED((N, D), jnp.float32),
            pltpu.SemaphoreType.DMA,
            pltpu.SemaphoreType.DMA,
        ],
    )(_msgpass_body)
    return k(y, src_r, dst_r, zeros)


def _pool_body(x_hbm, batch_hbm, zero_hbm, out_hbm, idx_v, rows_v, pool_sh, sem):
    c = lax.axis_index("c")
    s = lax.axis_index("s")
    wid = c * _NS + s

    @pl.when(s == 0)
    def _():
        pltpu.sync_copy(zero_hbm.at[pl.ds(0, G)], pool_sh)
    plsc.subcore_barrier()

    for j in range(_RJ):
        ch = wid + j * _NW

        @pl.when(ch < _NRCH)
        def _():
            base = ch * _KC
            pltpu.sync_copy(x_hbm.at[pl.ds(base, _KC)], rows_v)
            pltpu.sync_copy(batch_hbm.at[pl.ds(base, _KC)], idx_v)
            pltpu.sync_copy(rows_v, pool_sh.at[idx_v], add=True)
    plsc.subcore_barrier()

    @pl.when(s == 0)
    def _():
        pltpu.sync_copy(pool_sh, out_hbm.at[c])


def _sc_pool(x3, batch, zeros):
    k = functools.partial(
        pl.kernel,
        out_type=jax.ShapeDtypeStruct((_NC, G, D), jnp.float32),
        mesh=_sc_mesh(),
        scratch_types=[
            pltpu.VMEM((_KC,), jnp.int32),
            pltpu.VMEM((_KC, D), jnp.float32),
            pltpu.VMEM_SHARED((G, D), jnp.float32),
            pltpu.SemaphoreType.DMA,
        ],
    )(_pool_body)
    return k(x3, batch, zeros)


def _fused_body(has_w, y_ref, s_ref, e_ref, b_ref, g_ref, bt_ref, *rest):
    if has_w:
        w_ref, o_ref, h_ref, cs_ref, css_ref = rest
    else:
        o_ref, h_ref, cs_ref, css_ref = rest
    p = pl.program_id(0)
    i = pl.program_id(1)

    @pl.when(p == 0)
    def _():
        h = e_ref[...] * y_ref[...] + s_ref[0] + s_ref[1] + b_ref[...]
        h_ref[pl.ds(i * _RB, _RB), :] = h
        cs = jnp.sum(h, axis=0, keepdims=True)
        css = jnp.sum(h * h, axis=0, keepdims=True)

        @pl.when(i == 0)
        def _():
            cs_ref[...] = cs
            css_ref[...] = css

        @pl.when(i > 0)
        def _():
            cs_ref[...] += cs
            css_ref[...] += css

    @pl.when(p == 1)
    def _():
        mean = cs_ref[...] * (1.0 / N)
        var = css_ref[...] * (1.0 / N) - mean * mean
        inv = lax.rsqrt(var + 1e-5)
        xb = (h_ref[pl.ds(i * _RB, _RB), :] - mean) * (inv * g_ref[...]) \
            + bt_ref[...]
        xb = _lrelu(xb)
        if has_w:
            o_ref[...] = jnp.dot(xb, w_ref[...],
                                 preferred_element_type=jnp.float32)
        else:
            o_ref[...] = xb


def _tc_layer(y, s2, epsp1, b, gamma, beta, w_next=None):
    vec_spec = pl.BlockSpec((1, D), lambda p, i: (0, 0))
    in_specs = [pl.BlockSpec((_RB, D), lambda p, i: (i, 0)),
                pl.BlockSpec((_NC, _RB, D), lambda p, i: (0, i * (1 - p), 0)),
                vec_spec, vec_spec, vec_spec, vec_spec]
    args = [y, s2, epsp1, b, gamma, beta]
    if w_next is not None:
        in_specs.append(pl.BlockSpec((D, D), lambda p, i: (0, 0)))
        args.append(w_next)
    return pl.pallas_call(
        functools.partial(_fused_body, w_next is not None),
        grid=(2, _NB),
        in_specs=in_specs,
        out_specs=pl.BlockSpec((_RB, D), lambda p, i: (i, 0)),
        out_shape=jax.ShapeDtypeStruct((N, D), jnp.float32),
        scratch_shapes=[
            pltpu.VMEM((N, D), jnp.float32),
            pltpu.VMEM((1, D), jnp.float32),
            pltpu.VMEM((1, D), jnp.float32),
        ],
    )(*args)


def _head_body(p_ref, ls_ref, lw_ref, lb_ref, w1_ref, b1_ref, w2_ref, b2_ref,
               out_ref):
    pooled = p_ref[0] + p_ref[1]
    ls_h = jnp.dot(ls_ref[...], lw_ref[...],
                   preferred_element_type=jnp.float32) + lb_ref[...]
    w1 = w1_ref[...]
    o1 = (jnp.dot(pooled, w1[:D, :], preferred_element_type=jnp.float32)
          + jnp.dot(ls_h, w1[D:, :], preferred_element_type=jnp.float32)
          + b1_ref[...])
    o1 = _lrelu(o1)
    out_ref[...] = jnp.dot(o1, w2_ref[...],
                           preferred_element_type=jnp.float32) + b2_ref[...]


def _tc_head(pooled2, ls, linh_W, linh_b, cls_W1, cls_b1, cls_W2, cls_b2):
    return pl.pallas_call(
        _head_body,
        out_shape=jax.ShapeDtypeStruct((G, C), jnp.float32),
    )(pooled2, ls, linh_W, linh_b.reshape(1, H), cls_W1,
      cls_b1.reshape(1, H), cls_W2, cls_b2.reshape(1, C))


def kernel(node_deg, edge_index, batch, ls, conv_W, conv_b, eps, bn_gamma,
           bn_beta, linh_W, linh_b, cls_W1, cls_b1, cls_W2, cls_b2):
    src_r = edge_index[0].reshape(E // _EC, _EC)
    dst_r = edge_index[1].reshape(E // _EC, _EC)
    zeros = jnp.zeros((N, D), jnp.float32)

    y = _sc_y0(node_deg.astype(jnp.int32), conv_W[0])
    for l in range(L):
        s2 = _sc_msgpass(y, src_r, dst_r, zeros)
        epsp1 = jnp.full((1, D), 1.0, jnp.float32) + eps[l]
        w_next = conv_W[l + 1] if l + 1 < L else None
        y = _tc_layer(y, s2, epsp1, conv_b[l].reshape(1, D),
                      bn_gamma[l].reshape(1, D), bn_beta[l].reshape(1, D),
                      w_next)

    pooled2 = _sc_pool(y, batch.astype(jnp.int32), zeros)
    return _tc_head(pooled2, ls, linh_W, linh_b, cls_W1, cls_b1, cls_W2,
                    cls_b2)


# speedup vs baseline: 11.5972x; 1.0108x over previous
import functools

import jax
import jax.numpy as jnp
from jax import lax
from jax.experimental import pallas as pl
from jax.experimental.pallas import tpu as pltpu
from jax.experimental.pallas import tpu_sc as plsc

N = 10000
E = 320000
D = 128
G = 64
LS = 64
H = 128
C = 10
L = 3

_NC = 2
_NS = 16
_NW = _NC * _NS
_EC = 125
_EPW = E // _NW
_NECH = _EPW // _EC
_PCH = 40
_KC = 80
_NRCH = N // _KC
_RJ = -(-_NRCH // _NW)
_RJS = -(-_NRCH // _NS)

_RB = 1000
_NB = N // _RB


def _lrelu(x):
    return jnp.where(x >= 0, x, 0.01 * x)


def _sc_mesh():
    return plsc.VectorSubcoreMesh(core_axis_name="c", subcore_axis_name="s")


def _y0_body(deg_hbm, w_hbm, out_hbm, idx_v, rows_v, sem):
    c = lax.axis_index("c")
    s = lax.axis_index("s")
    wid = c * _NS + s
    for j in range(_RJ):
        ch = wid + j * _NW

        @pl.when(ch < _NRCH)
        def _():
            base = ch * _KC
            pltpu.sync_copy(deg_hbm.at[pl.ds(base, _KC)], idx_v)
            pltpu.async_copy(w_hbm.at[idx_v], rows_v, sem).wait()
            pltpu.sync_copy(rows_v, out_hbm.at[pl.ds(base, _KC)])


def _sc_y0(node_deg, w0):
    k = functools.partial(
        pl.kernel,
        out_type=jax.ShapeDtypeStruct((N, D), jnp.float32),
        mesh=_sc_mesh(),
        scratch_types=[
            pltpu.VMEM((_KC,), jnp.int32),
            pltpu.VMEM((_KC, D), jnp.float32),
            pltpu.SemaphoreType.DMA,
        ],
    )(_y0_body)
    return k(node_deg, w0)


def _msgpass_body(y_hbm, srcr_hbm, dstr_hbm, zero_hbm, out_hbm,
                  src2_v, dst2_v, buf0, buf1, acc_sh, gsem, ssem):
    c = lax.axis_index("c")
    s = lax.axis_index("s")
    wid = c * _NS + s

    for j in range(_RJS):
        ch = s + j * _NS

        @pl.when(ch < _NRCH)
        def _():
            base = ch * _KC
            pltpu.sync_copy(zero_hbm.at[pl.ds(base, _KC)],
                            acc_sh.at[pl.ds(base, _KC)])
    plsc.subcore_barrier()

    def gather(i, buf):
        pltpu.async_copy(y_hbm.at[src2_v.at[i]], buf, gsem)

    def drain_g(i, buf):
        pltpu.make_async_copy(y_hbm.at[src2_v.at[i]], buf, gsem).wait()

    def scat(i, buf):
        pltpu.async_copy(buf, acc_sh.at[dst2_v.at[i]], ssem, add=True)

    def drain_s(i, buf):
        pltpu.make_async_copy(buf, acc_sh.at[dst2_v.at[i]], ssem).wait()

    for p in range(_NECH // _PCH):
        pltpu.sync_copy(
            srcr_hbm.at[pl.ds(wid * _NECH + p * _PCH, _PCH)], src2_v)
        pltpu.sync_copy(
            dstr_hbm.at[pl.ds(wid * _NECH + p * _PCH, _PCH)], dst2_v)

        gather(0, buf0)

        def body(i, carry):
            c0 = 2 * i

            @pl.when(i > 0)
            def _():
                drain_s(c0 - 1, buf1)
            gather(c0 + 1, buf1)
            drain_g(c0, buf0)
            scat(c0, buf0)

            drain_s(c0, buf0)

            @pl.when(c0 + 2 < _PCH)
            def _():
                gather(c0 + 2, buf0)
            drain_g(c0 + 1, buf1)
            scat(c0 + 1, buf1)
            return carry

        lax.fori_loop(0, _PCH // 2, body, 0)
        drain_s(_PCH - 1, buf1)
    plsc.subcore_barrier()

    for j in range(_RJS):
        ch = s + j * _NS

        @pl.when(ch < _NRCH)
        def _():
            base = ch * _KC
            pltpu.sync_copy(acc_sh.at[pl.ds(base, _KC)],
                            out_hbm.at[c, pl.ds(base, _KC)])


def _sc_msgpass(y, src_r, dst_r, zeros):
    k = functools.partial(
        pl.kernel,
        out_type=jax.ShapeDtypeStruct((_NC, N, D), jnp.float32),
        mesh=_sc_mesh(),
        scratch_types=[
            pltpu.VMEM((_PCH, _EC), jnp.int32),
            pltpu.VMEM((_PCH, _EC), jnp.int32),
            pltpu.VMEM((_EC, D), jnp.float32),
            pltpu.VMEM((_EC, D), jnp.float32),
            pltpu.VMEM_SHARED((N, D), jnp.float32),
            pltpu.SemaphoreType.DMA,
            pltpu.SemaphoreType.DMA,
        ],
    )(_msgpass_body)
    return k(y, src_r, dst_r, zeros)


def _fused_body(has_w, y_ref, s_ref, e_ref, b_ref, g_ref, bt_ref, *rest):
    if has_w:
        w_ref, o_ref, h_ref, cs_ref, css_ref = rest
    else:
        o_ref, h_ref, cs_ref, css_ref = rest
    p = pl.program_id(0)
    i = pl.program_id(1)

    @pl.when(p == 0)
    def _():
        h = e_ref[...] * y_ref[...] + s_ref[0] + s_ref[1] + b_ref[...]
        h_ref[pl.ds(i * _RB, _RB), :] = h
        cs = jnp.sum(h, axis=0, keepdims=True)
        css = jnp.sum(h * h, axis=0, keepdims=True)

        @pl.when(i == 0)
        def _():
            cs_ref[...] = cs
            css_ref[...] = css

        @pl.when(i > 0)
        def _():
            cs_ref[...] += cs
            css_ref[...] += css

    @pl.when(p == 1)
    def _():
        mean = cs_ref[...] * (1.0 / N)
        var = css_ref[...] * (1.0 / N) - mean * mean
        inv = lax.rsqrt(var + 1e-5)
        xb = (h_ref[pl.ds(i * _RB, _RB), :] - mean) * (inv * g_ref[...]) \
            + bt_ref[...]
        xb = _lrelu(xb)
        if has_w:
            o_ref[...] = jnp.dot(xb, w_ref[...],
                                 preferred_element_type=jnp.float32)
        else:
            o_ref[...] = xb


def _tc_layer(y, s2, epsp1, b, gamma, beta, w_next=None):
    vec_spec = pl.BlockSpec((1, D), lambda p, i: (0, 0))
    in_specs = [pl.BlockSpec((_RB, D), lambda p, i: (i, 0)),
                pl.BlockSpec((_NC, _RB, D), lambda p, i: (0, i * (1 - p), 0)),
                vec_spec, vec_spec, vec_spec, vec_spec]
    args = [y, s2, epsp1, b, gamma, beta]
    if w_next is not None:
        in_specs.append(pl.BlockSpec((D, D), lambda p, i: (0, 0)))
        args.append(w_next)
    return pl.pallas_call(
        functools.partial(_fused_body, w_next is not None),
        grid=(2, _NB),
        in_specs=in_specs,
        out_specs=pl.BlockSpec((_RB, D), lambda p, i: (i, 0)),
        out_shape=jax.ShapeDtypeStruct((N, D), jnp.float32),
        scratch_shapes=[
            pltpu.VMEM((N, D), jnp.float32),
            pltpu.VMEM((1, D), jnp.float32),
            pltpu.VMEM((1, D), jnp.float32),
        ],
    )(*args)


def _pool_head_body(x_ref, b3_ref, ls_ref, lw_ref, lb_ref, w1_ref, b1_ref,
                    w2_ref, b2_ref, out_ref, pool_ref):
    i = pl.program_id(0)
    bvec = b3_ref[...].reshape(1, _RB)
    onehot_t = (lax.broadcasted_iota(jnp.int32, (G, _RB), 0)
                == bvec).astype(jnp.float32)
    contrib = jnp.dot(onehot_t, x_ref[...], preferred_element_type=jnp.float32)

    @pl.when(i == 0)
    def _():
        pool_ref[...] = contrib

    @pl.when(i > 0)
    def _():
        pool_ref[...] += contrib

    @pl.when(i == _NB - 1)
    def _():
        pooled = pool_ref[...]
        ls_h = jnp.dot(ls_ref[...], lw_ref[...],
                       preferred_element_type=jnp.float32) + lb_ref[...]
        w1 = w1_ref[...]
        o1 = (jnp.dot(pooled, w1[:D, :], preferred_element_type=jnp.float32)
              + jnp.dot(ls_h, w1[D:, :], preferred_element_type=jnp.float32)
              + b1_ref[...])
        o1 = _lrelu(o1)
        out_ref[...] = jnp.dot(o1, w2_ref[...],
                               preferred_element_type=jnp.float32) + b2_ref[...]


def _tc_pool_head(x3, batch3, ls, linh_W, linh_b, cls_W1, cls_b1, cls_W2,
                  cls_b2):
    def full(shape):
        return pl.BlockSpec(shape, lambda i: tuple(0 for _ in shape))

    return pl.pallas_call(
        _pool_head_body,
        grid=(_NB,),
        in_specs=[
            pl.BlockSpec((_RB, D), lambda i: (i, 0)),
            pl.BlockSpec((1, 1, _RB), lambda i: (i, 0, 0)),
            full((G, LS)), full((LS, H)), full((1, H)), full((2 * D, H)),
            full((1, H)), full((H, C)), full((1, C)),
        ],
        out_specs=pl.BlockSpec((G, C), lambda i: (0, 0)),
        out_shape=jax.ShapeDtypeStruct((G, C), jnp.float32),
        scratch_shapes=[pltpu.VMEM((G, D), jnp.float32)],
    )(x3, batch3, ls, linh_W, linh_b.reshape(1, H), cls_W1,
      cls_b1.reshape(1, H), cls_W2, cls_b2.reshape(1, C))


def kernel(node_deg, edge_index, batch, ls, conv_W, conv_b, eps, bn_gamma,
           bn_beta, linh_W, linh_b, cls_W1, cls_b1, cls_W2, cls_b2):
    src_r = edge_index[0].reshape(E // _EC, _EC)
    dst_r = edge_index[1].reshape(E // _EC, _EC)
    zeros = jnp.zeros((N, D), jnp.float32)

    y = _sc_y0(node_deg.astype(jnp.int32), conv_W[0])
    for l in range(L):
        s2 = _sc_msgpass(y, src_r, dst_r, zeros)
        epsp1 = jnp.full((1, D), 1.0, jnp.float32) + eps[l]
        w_next = conv_W[l + 1] if l + 1 < L else None
        y = _tc_layer(y, s2, epsp1, conv_b[l].reshape(1, D),
                      bn_gamma[l].reshape(1, D), bn_beta[l].reshape(1, D),
                      w_next)

    batch3 = batch.astype(jnp.int32).reshape(_NB, 1, _RB)
    return _tc_pool_head(y, batch3, ls, linh_W, linh_b, cls_W1, cls_b1,
                         cls_W2, cls_b2)
